# baseline (speedup 1.0000x reference)
import functools

import numpy as np
import jax
import jax.numpy as jnp
from jax import lax
from jax.experimental import pallas as pl
from jax.experimental.pallas import tpu as pltpu
from jax.experimental.pallas import tpu_sc as plsc

_BATCH = 4
_MAX_LEN = 2048
_D = 1024

_NC = 2
_NS = 16
_NW = _NC * _NS
_L = 16

_P_W = _MAX_LEN // _NW
_PER_W = _BATCH * _P_W
_CH = 16
_NQ = _P_W // _CH
_NCH = _BATCH * _NQ
_NR = 3


def _positional_encoding() -> np.ndarray:
    pos = np.arange(_MAX_LEN, dtype=np.float32)[:, None]
    i = np.arange(0, _D, 2, dtype=np.float32)
    div = np.exp(-np.log(10000.0) * i / _D)
    pe = np.zeros((_MAX_LEN, _D), dtype=np.float32)
    pe[:, 0::2] = np.sin(pos * div)
    pe[:, 1::2] = np.cos(pos * div)
    return pe


_POS = _positional_encoding()

_mesh = plsc.VectorSubcoreMesh(core_axis_name="c", subcore_axis_name="s")


@functools.partial(
    pl.kernel,
    mesh=_mesh,
    out_type=jax.ShapeDtypeStruct((_BATCH * _MAX_LEN, _D), jnp.float32),
    scratch_types=(
        [pltpu.VMEM((_PER_W,), jnp.int32),
         pltpu.VMEM((_P_W, _D), jnp.float32)]
        + [pltpu.VMEM((_CH, _D), jnp.float32)] * _NR
        + [pltpu.SemaphoreType.DMA] * (1 + 2 * _NR)
    ),
)
def _emb_kernel(x_hbm, pos_hbm, table_hbm, out_hbm, idx_v, pos_v,
                rows0, rows1, rows2,
                hsem, gsem0, gsem1, gsem2, osem0, osem1, osem2):
    rows = (rows0, rows1, rows2)
    gsem = (gsem0, gsem1, gsem2)
    osem = (osem0, osem1, osem2)

    wid = lax.axis_index("s") * _NC + lax.axis_index("c")
    p0 = wid * _P_W

    hold_cp = pltpu.async_copy(pos_hbm.at[pl.ds(p0, _P_W)], pos_v, hsem)

    for b in range(_BATCH):
        pltpu.sync_copy(x_hbm.at[pl.ds(b * _MAX_LEN + p0, _P_W)],
                        idx_v.at[pl.ds(b * _P_W, _P_W)])

    def fire_gather(c):
        b, q = divmod(c, _NQ)
        return pltpu.async_copy(
            table_hbm.at[idx_v.at[pl.ds(b * _P_W + q * _CH, _CH)]],
            rows[c % _NR], gsem[c % _NR])

    g_cp = [None] * _NCH
    o_cp = [None] * _NCH
    g_cp[0] = fire_gather(0)
    g_cp[1] = fire_gather(1)

    for c in range(_NCH):
        b, q = divmod(c, _NQ)
        rb = rows[c % _NR]
        g_cp[c].wait()
        if c == 0:
            hold_cp.wait()

        def row(i, carry, rb=rb, q=q):
            for j in range(_D // _L):
                sl = pl.ds(j * _L, _L)
                plsc.addupdate(rb.at[i, sl], pos_v[q * _CH + i, sl])
            return carry

        lax.fori_loop(0, _CH, row, 0)

        o_cp[c] = pltpu.async_copy(
            rb, out_hbm.at[pl.ds(b * _MAX_LEN + p0 + q * _CH, _CH)],
            osem[c % _NR])
        if c + 2 < _NCH:
            if c >= 1:
                o_cp[c - 1].wait()
            g_cp[c + 2] = fire_gather(c + 2)

    for c in range(_NCH - _NR, _NCH):
        o_cp[c].wait()


def kernel(x, start_token, end_token, table):
    batch, seq_len = x.shape
    out = _emb_kernel(x.reshape(-1), jnp.asarray(_POS), table)
    return out.reshape(batch, seq_len, _D)
